# baseline (speedup 1.0000x reference)
import functools

import jax
import jax.numpy as jnp
from jax import lax
from jax.experimental import pallas as pl
from jax.experimental.pallas import tpu as pltpu
from jax.experimental.pallas import tpu_sc as plsc

B, N, K = 128, 400, 3
H = 256
L = 2
NDIM = 2
BB = 4

BN = B * N
NC, NS = 2, 16
NW = NC * NS
RPW = BN // NW
C = 64
NCH = RPW // C


def _sc_gather_sum(x_hbm, ce0_hbm, ce1_hbm, ce2_hbm, out_hbm,
                   i0, i1, i2, r0, r1, r2, sem):
    wid = lax.axis_index("s") * NC + lax.axis_index("c")
    base0 = wid * RPW

    def chunk(j, carry):
        base = base0 + j * C
        pltpu.sync_copy(ce0_hbm.at[pl.ds(base, C)], i0)
        pltpu.sync_copy(ce1_hbm.at[pl.ds(base, C)], i1)
        pltpu.sync_copy(ce2_hbm.at[pl.ds(base, C)], i2)
        cp0 = pltpu.async_copy(x_hbm.at[i0], r0, sem)
        cp1 = pltpu.async_copy(x_hbm.at[i1], r1, sem)
        cp2 = pltpu.async_copy(x_hbm.at[i2], r2, sem)
        cp0.wait()
        cp1.wait()
        cp2.wait()

        def row(rr, c2):
            for h in range(H // 16):
                sl = pl.ds(h * 16, 16)
                r0[rr, sl] = r0[rr, sl] + r1[rr, sl] + r2[rr, sl]
            return c2

        lax.fori_loop(0, C, row, 0)
        pltpu.sync_copy(r0, out_hbm.at[pl.ds(base, C)])
        return carry

    lax.fori_loop(0, NCH, chunk, 0)


_gather_call = functools.partial(
    pl.kernel,
    mesh=plsc.VectorSubcoreMesh(core_axis_name="c", subcore_axis_name="s"),
    out_type=jax.ShapeDtypeStruct((BN, H), jnp.float32),
    scratch_types=[
        pltpu.VMEM((C,), jnp.int32),
        pltpu.VMEM((C,), jnp.int32),
        pltpu.VMEM((C,), jnp.int32),
        pltpu.VMEM((C, H), jnp.float32),
        pltpu.VMEM((C, H), jnp.float32),
        pltpu.VMEM((C, H), jnp.float32),
        pltpu.SemaphoreType.DMA,
    ],
)(_sc_gather_sum)


def _elu(v):
    return jnp.where(v > 0, v, jnp.exp(jnp.minimum(v, 0.0)) - 1.0)


def _sigmoid(v):
    return 0.5 + 0.5 * jnp.tanh(0.5 * v)


def _gru(x, sx, vwih_ref, whh_ref, bih, bhh):
    gi = jnp.dot(sx, vwih_ref[...], preferred_element_type=jnp.float32) + bih
    gh = jnp.dot(x, whh_ref[...], preferred_element_type=jnp.float32) + bhh
    r = _sigmoid(gi[:, :H] + gh[:, :H])
    z = _sigmoid(gi[:, H:2 * H] + gh[:, H:2 * H])
    n = jnp.tanh(gi[:, 2 * H:] + r * gh[:, 2 * H:])
    return (1.0 - z) * n + z * x


def _gates_body(x_ref, sx_ref, vwih_ref, whh_ref, bih_ref, bhh_ref, out_ref):
    x = x_ref[...].reshape(BB * N, H)
    sx = sx_ref[...].reshape(BB * N, H)
    xn = _gru(x, sx, vwih_ref, whh_ref, bih_ref[0], bhh_ref[0])
    out_ref[...] = xn.reshape(BB, N, H)


def _final_body(x_ref, sx_ref, vwih_ref, whh_ref, bih_ref, bhh_ref,
                nw1_ref, nb1_ref, nw2_ref, nb2_ref, rw1_ref, rb1_ref,
                rw2_ref, rb2_ref, out_ref):
    x = x_ref[...].reshape(BB * N, H)
    sx = sx_ref[...].reshape(BB * N, H)
    x = _gru(x, sx, vwih_ref, whh_ref, bih_ref[0], bhh_ref[0])
    h1 = _elu(jnp.dot(x, nw1_ref[...], preferred_element_type=jnp.float32)
              + nb1_ref[0])
    h2 = _elu(jnp.dot(h1, nw2_ref[...], preferred_element_type=jnp.float32)
              + nb2_ref[0])
    pooled = jnp.sum(h2.reshape(BB, N, H), axis=1)
    r1 = _elu(jnp.dot(pooled, rw1_ref[...], preferred_element_type=jnp.float32)
              + rb1_ref[0])
    out = jnp.dot(r1, rw2_ref[...], preferred_element_type=jnp.float32) \
        + rb2_ref[0]
    out_ref[...] = out.reshape(BB, 1, NDIM)


def _rep(shape):
    return pl.BlockSpec(shape, lambda b: (0,) * len(shape))


def _tc_gates(x, sx, vwih, whh_t, bih, bhh):
    grid_spec = pl.GridSpec(
        grid=(B // BB,),
        in_specs=[
            pl.BlockSpec((BB, N, H), lambda b: (b, 0, 0)),
            pl.BlockSpec((BB, N, H), lambda b: (b, 0, 0)),
            _rep((H, 3 * H)),
            _rep((H, 3 * H)),
            _rep((1, 3 * H)),
            _rep((1, 3 * H)),
        ],
        out_specs=pl.BlockSpec((BB, N, H), lambda b: (b, 0, 0)),
    )
    return pl.pallas_call(
        _gates_body,
        grid_spec=grid_spec,
        out_shape=jax.ShapeDtypeStruct((B, N, H), jnp.float32),
        compiler_params=pltpu.CompilerParams(
            dimension_semantics=("arbitrary",)),
    )(x, sx, vwih, whh_t, bih, bhh)


def _tc_final(x, sx, vwih, whh_t, bih, bhh, nw1_t, nb1, nw2_t, nb2,
              rw1_t, rb1, rw2_t, rb2):
    grid_spec = pl.GridSpec(
        grid=(B // BB,),
        in_specs=[
            pl.BlockSpec((BB, N, H), lambda b: (b, 0, 0)),
            pl.BlockSpec((BB, N, H), lambda b: (b, 0, 0)),
            _rep((H, 3 * H)),
            _rep((H, 3 * H)),
            _rep((1, 3 * H)),
            _rep((1, 3 * H)),
            _rep((H, H)),
            _rep((1, H)),
            _rep((H, H)),
            _rep((1, H)),
            _rep((H, H)),
            _rep((1, H)),
            _rep((H, NDIM)),
            _rep((1, NDIM)),
        ],
        out_specs=pl.BlockSpec((BB, 1, NDIM), lambda b: (b, 0, 0)),
    )
    return pl.pallas_call(
        _final_body,
        grid_spec=grid_spec,
        out_shape=jax.ShapeDtypeStruct((B, 1, NDIM), jnp.float32),
        compiler_params=pltpu.CompilerParams(
            dimension_semantics=("arbitrary",)),
    )(x, sx, vwih, whh_t, bih, bhh, nw1_t, nb1, nw2_t, nb2,
      rw1_t, rb1, rw2_t, rb2)


@jax.jit
def kernel(node_features, edge_index, weight, w_ih, w_hh, b_ih, b_hh,
           net_w1, net_b1, net_w2, net_b2, ro_w1, ro_b1, ro_w2, ro_b2):
    wih_t = w_ih.T
    whh_t = w_hh.T
    vw = jnp.einsum('lhj,jg->lhg', weight, wih_t)
    nw1_t = net_w1.T
    nw2_t = net_w2.T
    rw1_t = ro_w1.T
    rw2_t = ro_w2.T
    bih = b_ih.reshape(1, 3 * H)
    bhh = b_hh.reshape(1, 3 * H)
    nb1 = net_b1.reshape(1, H)
    nb2 = net_b2.reshape(1, H)
    rb1 = ro_b1.reshape(1, H)
    rb2 = ro_b2.reshape(1, NDIM)

    offs = jnp.arange(B, dtype=jnp.int32)[:, None, None] * N
    ce = (edge_index + offs).reshape(BN, K)
    ce0 = ce[:, 0].reshape(BN)
    ce1 = ce[:, 1].reshape(BN)
    ce2 = ce[:, 2].reshape(BN)

    x0 = node_features.reshape(BN, H)
    sx0 = _gather_call(x0, ce0, ce1, ce2)
    x1 = _tc_gates(node_features, sx0.reshape(B, N, H),
                   vw[0], whh_t, bih, bhh)
    sx1 = _gather_call(x1.reshape(BN, H), ce0, ce1, ce2)
    out = _tc_final(x1, sx1.reshape(B, N, H), vw[1], whh_t, bih, bhh,
                    nw1_t, nb1, nw2_t, nb2, rw1_t, rb1, rw2_t, rb2)
    return out.reshape(B, NDIM)


# speedup vs baseline: 2.1180x; 2.1180x over previous
import functools

import jax
import jax.numpy as jnp
from jax.experimental import pallas as pl
from jax.experimental.pallas import tpu as pltpu

B, N, K = 128, 400, 3
H = 256
L = 2
NDIM = 2
BB = 4


def _elu(v):
    return jnp.where(v > 0, v, jnp.exp(jnp.minimum(v, 0.0)) - 1.0)


def _sigmoid(v):
    return 0.5 + 0.5 * jnp.tanh(0.5 * v)


def _body(x_ref, e_ref, wcat_ref, wih_ref, bih_ref, bhh_ref,
          nw1_ref, nb1_ref, nw2_ref, nb2_ref, rw1_ref, rb1_ref, rw2_ref,
          rb2_ref, out_ref):
    x = x_ref[...].reshape(BB * N, H)

    iota = jax.lax.broadcasted_iota(jnp.int32, (N, N), 1)
    adj = []
    for g in range(BB):
        e = e_ref[g]
        a = (e[:, 0:1] == iota).astype(jnp.float32)
        for k in range(1, K):
            a = a + (e[:, k:k + 1] == iota).astype(jnp.float32)
        adj.append(a)

    bih = bih_ref[0]
    bhh = bhh_ref[0]
    for i in range(L):
        mg = jnp.dot(x, wcat_ref[i], preferred_element_type=jnp.float32)
        m = mg[:, :H]
        gh = mg[:, H:] + bhh
        s = jnp.concatenate(
            [jnp.dot(adj[g], m[g * N:(g + 1) * N],
                     preferred_element_type=jnp.float32)
             for g in range(BB)], axis=0)
        gi = jnp.dot(s, wih_ref[...], preferred_element_type=jnp.float32) + bih
        r = _sigmoid(gi[:, :H] + gh[:, :H])
        z = _sigmoid(gi[:, H:2 * H] + gh[:, H:2 * H])
        n = jnp.tanh(gi[:, 2 * H:] + r * gh[:, 2 * H:])
        x = (1.0 - z) * n + z * x

    h1 = _elu(jnp.dot(x, nw1_ref[...], preferred_element_type=jnp.float32)
              + nb1_ref[0])
    h2 = _elu(jnp.dot(h1, nw2_ref[...], preferred_element_type=jnp.float32)
              + nb2_ref[0])
    pooled = jnp.sum(h2.reshape(BB, N, H), axis=1)
    r1 = _elu(jnp.dot(pooled, rw1_ref[...], preferred_element_type=jnp.float32)
              + rb1_ref[0])
    out = jnp.dot(r1, rw2_ref[...], preferred_element_type=jnp.float32) \
        + rb2_ref[0]
    out_ref[...] = out.reshape(BB, 1, NDIM)


@jax.jit
def kernel(node_features, edge_index, weight, w_ih, w_hh, b_ih, b_hh,
           net_w1, net_b1, net_w2, net_b2, ro_w1, ro_b1, ro_w2, ro_b2):
    wih_t = w_ih.T
    whh_t = w_hh.T
    wcat = jnp.concatenate(
        [weight, jnp.broadcast_to(whh_t[None], (L, H, 3 * H))], axis=2)
    nw1_t = net_w1.T
    nw2_t = net_w2.T
    rw1_t = ro_w1.T
    rw2_t = ro_w2.T
    bih = b_ih.reshape(1, 3 * H)
    bhh = b_hh.reshape(1, 3 * H)
    nb1 = net_b1.reshape(1, H)
    nb2 = net_b2.reshape(1, H)
    rb1 = ro_b1.reshape(1, H)
    rb2 = ro_b2.reshape(1, NDIM)

    rep = lambda shape: pl.BlockSpec(shape, lambda b: (0,) * len(shape))
    grid_spec = pl.GridSpec(
        grid=(B // BB,),
        in_specs=[
            pl.BlockSpec((BB, N, H), lambda b: (b, 0, 0)),
            pl.BlockSpec((BB, N, K), lambda b: (b, 0, 0)),
            rep((L, H, 4 * H)),
            rep((H, 3 * H)),
            rep((1, 3 * H)),
            rep((1, 3 * H)),
            rep((H, H)),
            rep((1, H)),
            rep((H, H)),
            rep((1, H)),
            rep((H, H)),
            rep((1, H)),
            rep((H, NDIM)),
            rep((1, NDIM)),
        ],
        out_specs=pl.BlockSpec((BB, 1, NDIM), lambda b: (b, 0, 0)),
    )
    out = pl.pallas_call(
        _body,
        grid_spec=grid_spec,
        out_shape=jax.ShapeDtypeStruct((B, 1, NDIM), jnp.float32),
        compiler_params=pltpu.CompilerParams(
            dimension_semantics=("arbitrary",),
        ),
    )(node_features, edge_index, wcat, wih_t, bih, bhh,
      nw1_t, nb1, nw2_t, nb2, rw1_t, rb1, rw2_t, rb2)
    return out.reshape(B, NDIM)


# speedup vs baseline: 2.1952x; 1.0365x over previous
import functools

import jax
import jax.numpy as jnp
from jax.experimental import pallas as pl
from jax.experimental.pallas import tpu as pltpu

B, N, K = 128, 400, 3
H = 256
L = 2
NDIM = 2
BB = 8


def _elu(v):
    return jnp.where(v > 0, v, jnp.exp(jnp.minimum(v, 0.0)) - 1.0)


def _sigmoid(v):
    return 0.5 + 0.5 * jnp.tanh(0.5 * v)


def _body(x_ref, e_ref, wcat_ref, wih_ref, bih_ref, bhh_ref,
          nw1_ref, nb1_ref, nw2_ref, nb2_ref, rw1_ref, rb1_ref, rw2_ref,
          rb2_ref, out_ref):
    x = x_ref[...].reshape(BB * N, H)

    iota = jax.lax.broadcasted_iota(jnp.int32, (N, N), 1)
    adj = []
    for g in range(BB):
        e = e_ref[g]
        a = (e[:, 0:1] == iota).astype(jnp.float32)
        for k in range(1, K):
            a = a + (e[:, k:k + 1] == iota).astype(jnp.float32)
        adj.append(a)

    bih = bih_ref[0]
    bhh = bhh_ref[0]
    for i in range(L):
        mg = jnp.dot(x, wcat_ref[i], preferred_element_type=jnp.float32)
        m = mg[:, :H]
        gh = mg[:, H:] + bhh
        s = jnp.concatenate(
            [jnp.dot(adj[g], m[g * N:(g + 1) * N],
                     preferred_element_type=jnp.float32)
             for g in range(BB)], axis=0)
        gi = jnp.dot(s, wih_ref[...], preferred_element_type=jnp.float32) + bih
        r = _sigmoid(gi[:, :H] + gh[:, :H])
        z = _sigmoid(gi[:, H:2 * H] + gh[:, H:2 * H])
        n = jnp.tanh(gi[:, 2 * H:] + r * gh[:, 2 * H:])
        x = (1.0 - z) * n + z * x

    h1 = _elu(jnp.dot(x, nw1_ref[...], preferred_element_type=jnp.float32)
              + nb1_ref[0])
    h2 = _elu(jnp.dot(h1, nw2_ref[...], preferred_element_type=jnp.float32)
              + nb2_ref[0])
    pooled = jnp.sum(h2.reshape(BB, N, H), axis=1)
    r1 = _elu(jnp.dot(pooled, rw1_ref[...], preferred_element_type=jnp.float32)
              + rb1_ref[0])
    out = jnp.dot(r1, rw2_ref[...], preferred_element_type=jnp.float32) \
        + rb2_ref[0]
    out_ref[...] = out.reshape(BB, 1, NDIM)


@jax.jit
def kernel(node_features, edge_index, weight, w_ih, w_hh, b_ih, b_hh,
           net_w1, net_b1, net_w2, net_b2, ro_w1, ro_b1, ro_w2, ro_b2):
    wih_t = w_ih.T
    whh_t = w_hh.T
    wcat = jnp.concatenate(
        [weight, jnp.broadcast_to(whh_t[None], (L, H, 3 * H))], axis=2)
    nw1_t = net_w1.T
    nw2_t = net_w2.T
    rw1_t = ro_w1.T
    rw2_t = ro_w2.T
    bih = b_ih.reshape(1, 3 * H)
    bhh = b_hh.reshape(1, 3 * H)
    nb1 = net_b1.reshape(1, H)
    nb2 = net_b2.reshape(1, H)
    rb1 = ro_b1.reshape(1, H)
    rb2 = ro_b2.reshape(1, NDIM)

    rep = lambda shape: pl.BlockSpec(shape, lambda b: (0,) * len(shape))
    grid_spec = pl.GridSpec(
        grid=(B // BB,),
        in_specs=[
            pl.BlockSpec((BB, N, H), lambda b: (b, 0, 0)),
            pl.BlockSpec((BB, N, K), lambda b: (b, 0, 0)),
            rep((L, H, 4 * H)),
            rep((H, 3 * H)),
            rep((1, 3 * H)),
            rep((1, 3 * H)),
            rep((H, H)),
            rep((1, H)),
            rep((H, H)),
            rep((1, H)),
            rep((H, H)),
            rep((1, H)),
            rep((H, NDIM)),
            rep((1, NDIM)),
        ],
        out_specs=pl.BlockSpec((BB, 1, NDIM), lambda b: (b, 0, 0)),
    )
    out = pl.pallas_call(
        _body,
        grid_spec=grid_spec,
        out_shape=jax.ShapeDtypeStruct((B, 1, NDIM), jnp.float32),
        compiler_params=pltpu.CompilerParams(
            dimension_semantics=("arbitrary",),
        ),
    )(node_features, edge_index, wcat, wih_t, bih, bhh,
      nw1_t, nb1, nw2_t, nb2, rw1_t, rb1, rw2_t, rb2)
    return out.reshape(B, NDIM)
